# baseline (speedup 1.0000x reference)
import jax
import jax.numpy as jnp
from jax.experimental import pallas as pl
from jax.experimental.pallas import tpu as pltpu

_B_T2I = 2
_B_LM = 1
_B_MMU = 1
_S = 2048
_IMG_BEGIN, _IMG_END = 128, 1152
_PAD_BEGIN_ENDS = ((0, 80), (0, 100), (0, 110), (0, 0))

_BQ = 512


def _mask_kernel(pads_ref, out_ref):
    b = pl.program_id(0)
    qi = pl.program_id(1)
    pb = pads_ref[b, 0]
    pe = pads_ref[b, 1]
    q = qi * _BQ + jax.lax.broadcasted_iota(jnp.int32, (_BQ, _S), 0)
    kv = jax.lax.broadcasted_iota(jnp.int32, (_BQ, _S), 1)
    is_pad = (kv >= pb) & (kv < pe)
    causal = (~is_pad & (q >= kv)) | (q == kv)
    full = (q >= _IMG_BEGIN) & (q < _IMG_END) & (kv >= _IMG_BEGIN) & (kv < _IMG_END)
    out_ref[0] = causal | full


def kernel(sequence):
    B, S = sequence.shape
    pads = jnp.asarray(_PAD_BEGIN_ENDS, dtype=jnp.int32)
    return pl.pallas_call(
        _mask_kernel,
        grid=(B, S // _BQ),
        in_specs=[pl.BlockSpec(memory_space=pltpu.SMEM)],
        out_specs=pl.BlockSpec((1, _BQ, S), lambda b, qi: (b, qi, 0)),
        out_shape=jax.ShapeDtypeStruct((B, S, S), jnp.bool_),
    )(pads)


# speedup vs baseline: 1.1220x; 1.1220x over previous
import jax
import jax.numpy as jnp
from jax.experimental import pallas as pl
from jax.experimental.pallas import tpu as pltpu

_B_T2I = 2
_B_LM = 1
_B_MMU = 1
_S = 2048
_IMG_BEGIN, _IMG_END = 128, 1152
_PAD_BEGIN_ENDS = ((0, 80), (0, 100), (0, 110), (0, 0))

_BQ = 512


def _mask_kernel(pads_ref, out_ref):
    b = pl.program_id(0)
    qi = pl.program_id(1)
    pe = pads_ref[b, 1]
    q = qi * _BQ + jax.lax.broadcasted_iota(jnp.int32, (_BQ, 1), 0)
    in_img = (q >= _IMG_BEGIN) & (q < _IMG_END)
    lo = jnp.minimum(q, pe)
    hi = jnp.where(in_img, _IMG_END, q + 1)
    kv = jax.lax.broadcasted_iota(jnp.int32, (_BQ, _S), 1)
    out_ref[0] = (kv >= lo) & (kv < hi)


def kernel(sequence):
    B, S = sequence.shape
    pads = jnp.asarray(_PAD_BEGIN_ENDS, dtype=jnp.int32)
    return pl.pallas_call(
        _mask_kernel,
        grid=(B, S // _BQ),
        in_specs=[pl.BlockSpec(memory_space=pltpu.SMEM)],
        out_specs=pl.BlockSpec((1, _BQ, S), lambda b, qi: (b, qi, 0)),
        out_shape=jax.ShapeDtypeStruct((B, S, S), jnp.bool_),
    )(pads)
